# baseline (speedup 1.0000x reference)
import functools

import jax
import jax.numpy as jnp
import numpy as np
from jax import lax
from jax.experimental import pallas as pl
from jax.experimental.pallas import tpu as pltpu
from jax.experimental.pallas import tpu_sc as plsc

B, N, D, V = 2, 2048, 768, 8192
MASK_ID = V
SUB, LANE = 16, 128
NO_REPLACE_PROB = np.float32(0.15)
RANDOM_TOKEN_PROB_EFF = np.float32(0.05 * (1.0 - 0.15))

TBLK = 256
VBLK = 2048
NTB, NVB = (B * N) // TBLK, V // VBLK


def _bitonic_argsort(v, ilin, li, si):
    idx = ilin
    for klog in range(1, 12):
        kbit = 1 << klog
        asc = (ilin & kbit) == 0
        for jlog in reversed(range(klog)):
            d = 1 << jlog
            if d < LANE:
                ax, sh = 1, d
                lower = (li & d) == 0
            else:
                ax, sh = 0, d // LANE
                lower = (si & sh) == 0
            pv = jnp.where(lower, jnp.roll(v, -sh, axis=ax), jnp.roll(v, sh, axis=ax))
            pidx = jnp.where(lower, jnp.roll(idx, -sh, axis=ax), jnp.roll(idx, sh, axis=ax))
            gtp = (v > pv) | ((v == pv) & (idx > pidx))
            take = gtp == (asc == lower)
            v = jnp.where(take, pv, v)
            idx = jnp.where(take, pidx, idx)
    return idx


def _mask_kernel(rt_ref, x_ref, perm_ref, s1_ref, s2_ref, rtok_ref,
                 mask_ref, ids_ref):
    li = lax.broadcasted_iota(jnp.int32, (SUB, LANE), 1)
    si = lax.broadcasted_iota(jnp.int32, (SUB, LANE), 0)
    ilin = si * LANE + li

    sidx = _bitonic_argsort(perm_ref[0], ilin, li, si)
    rt = rt_ref[0, 0, 0]
    numtok = jnp.maximum((1.0 - rt) * np.float32(N), np.float32(1.0))
    mask = sidx.astype(jnp.float32) < numtok

    msum = jnp.sum(mask.astype(jnp.float32))
    p1 = np.float32(N) - msum
    nm1 = jnp.maximum(msum * NO_REPLACE_PROB, np.float32(0.0))
    a2 = jnp.where(mask, s1_ref[0], np.float32(-1.0))
    sidx2 = _bitonic_argsort(a2, ilin, li, si)
    norep = mask & ((sidx2.astype(jnp.float32) - p1) < nm1)
    rep = mask & (~norep)

    rsum = jnp.sum(rep.astype(jnp.float32))
    p2 = np.float32(N) - rsum
    nm2 = jnp.maximum(rsum * RANDOM_TOKEN_PROB_EFF, np.float32(0.0))
    a3 = jnp.where(rep, s2_ref[0], np.float32(-1.0))
    sidx3 = _bitonic_argsort(a3, ilin, li, si)
    rndm = rep & ((sidx3.astype(jnp.float32) - p2) < nm2)
    rep_final = rep & (~rndm)

    x2 = jnp.where(rndm, rtok_ref[0], x_ref[0])
    ids_ref[0] = jnp.where(rep_final, MASK_ID, x2)
    mask_ref[0] = mask.astype(jnp.float32)


def _compute_masks(rt2, xr, pr, s1r, s2r, rtokr):
    blk = pl.BlockSpec((1, SUB, LANE), lambda b: (b, 0, 0))
    return pl.pallas_call(
        _mask_kernel,
        grid=(B,),
        in_specs=[pl.BlockSpec((1, 1, 1), lambda b: (b, 0, 0), memory_space=pltpu.SMEM),
                  blk, blk, blk, blk, blk],
        out_specs=[blk, blk],
        out_shape=[
            jax.ShapeDtypeStruct((B, SUB, LANE), jnp.float32),
            jax.ShapeDtypeStruct((B, SUB, LANE), jnp.int32),
        ],
    )(rt2, xr, pr, s1r, s2r, rtokr)


_SC_CORES, _SC_SUBCORES = 2, 16
_NW = _SC_CORES * _SC_SUBCORES


def _gather_body(bpw, table_hbm, idx_hbm, out_hbm, idx_v, rows_v, sem):
    wid = lax.axis_index("s") * _SC_CORES + lax.axis_index("c")
    base = wid * bpw
    pltpu.sync_copy(idx_hbm.at[pl.ds(base, bpw)], idx_v)
    pltpu.async_copy(table_hbm.at[idx_v], rows_v, sem).wait()
    pltpu.sync_copy(rows_v, out_hbm.at[pl.ds(base, bpw)])


def _gather_rows(table, ids_chunk):
    ntok = ids_chunk.shape[0]
    bpw = ntok // _NW
    mesh = plsc.VectorSubcoreMesh(core_axis_name="c", subcore_axis_name="s")
    k = pl.kernel(
        functools.partial(_gather_body, bpw),
        out_type=jax.ShapeDtypeStruct((ntok, D), jnp.float32),
        mesh=mesh,
        scratch_types=[
            pltpu.VMEM((bpw,), jnp.int32),
            pltpu.VMEM((bpw, D), jnp.float32),
            pltpu.SemaphoreType.DMA,
        ],
    )
    return k(table, ids_chunk)


CH = 4
TPC = (B * N) // CH
NTBC = TPC // TBLK


def _loss_chunk_kernel(first, last, *refs):
    if first:
        h_ref, w_ref, lab_ref, maskw_ref, out_ref, loss_acc, cnt_acc = refs
    else:
        h_ref, w_ref, lab_ref, maskw_ref, pin_ref, out_ref, loss_acc, cnt_acc = refs
    t = pl.program_id(0)
    logits = jnp.dot(h_ref[...].astype(jnp.bfloat16), w_ref[...],
                     preferred_element_type=jnp.float32)
    m = jnp.max(logits, axis=1, keepdims=True)
    s = jnp.sum(jnp.exp(logits - m), axis=1, keepdims=True)
    hit = lax.broadcasted_iota(jnp.int32, (TBLK, V), 1) == lab_ref[...]
    lab = jnp.sum(jnp.where(hit, logits, 0.0), axis=1, keepdims=True)
    tok_ll = lab - (jnp.log(s) + m)

    @pl.when(t == 0)
    def _zero():
        if first:
            loss_acc[0, 0] = 0.0
            cnt_acc[0, 0] = 0.0
        else:
            loss_acc[0, 0] = pin_ref[0, 0]
            cnt_acc[0, 0] = pin_ref[0, 1]

    loss_acc[0, 0] += jnp.sum(maskw_ref[...] * tok_ll)
    cnt_acc[0, 0] += jnp.sum(maskw_ref[...])

    @pl.when(t == NTBC - 1)
    def _out():
        if last:
            out_ref[...] = jnp.full((1, 1), -loss_acc[0, 0] / cnt_acc[0, 0],
                                    jnp.float32)
        else:
            out_ref[0, 0] = loss_acc[0, 0]
            out_ref[0, 1] = cnt_acc[0, 0]


def _masked_loss_chunk(h, w_bf, labels, maskw, partial_in, first, last):
    in_specs = [
        pl.BlockSpec((TBLK, D), lambda t: (t, 0)),
        pl.BlockSpec((D, V), lambda t: (0, 0)),
        pl.BlockSpec((TBLK, 1), lambda t: (t, 0)),
        pl.BlockSpec((TBLK, 1), lambda t: (t, 0)),
    ]
    args = [h, w_bf, labels, maskw]
    if not first:
        in_specs.append(pl.BlockSpec((1, 2), lambda t: (0, 0),
                                     memory_space=pltpu.SMEM))
        args.append(partial_in)
    if last:
        out_spec = pl.BlockSpec((1, 1), lambda t: (0, 0))
        out_shape = jax.ShapeDtypeStruct((1, 1), jnp.float32)
    else:
        out_spec = pl.BlockSpec((1, 2), lambda t: (0, 0),
                                memory_space=pltpu.SMEM)
        out_shape = jax.ShapeDtypeStruct((1, 2), jnp.float32)
    return pl.pallas_call(
        functools.partial(_loss_chunk_kernel, first, last),
        grid=(NTBC,),
        in_specs=in_specs,
        out_specs=out_spec,
        out_shape=out_shape,
        scratch_shapes=[
            pltpu.SMEM((1, 1), jnp.float32),
            pltpu.SMEM((1, 1), jnp.float32),
        ],
    )(*args)


def kernel(x, embed, W_out, rand_times, perm_rand, subset_rand1,
           subset_rand2, random_tokens):
    xr = x.reshape(B, SUB, LANE)
    pr = perm_rand.reshape(B, SUB, LANE)
    s1r = subset_rand1.reshape(B, SUB, LANE)
    s2r = subset_rand2.reshape(B, SUB, LANE)
    rtokr = random_tokens.reshape(B, SUB, LANE)
    rt2 = rand_times.reshape(B, 1, 1)

    mask_f, ids = _compute_masks(rt2, xr, pr, s1r, s2r, rtokr)
    ids_flat = ids.reshape(B * N)
    labels = x.reshape(B * N, 1)
    maskw = mask_f.reshape(B * N, 1)
    w_bf = W_out.astype(jnp.bfloat16)

    hs = [_gather_rows(embed, lax.slice(ids_flat, (c * TPC,), ((c + 1) * TPC,)))
          for c in range(CH)]
    partial = None
    for c in range(CH):
        sl = slice(c * TPC, (c + 1) * TPC)
        partial = _masked_loss_chunk(hs[c], w_bf, labels[sl], maskw[sl],
                                     partial, first=(c == 0), last=(c == CH - 1))
    return partial[0, 0]


# speedup vs baseline: 1.1081x; 1.1081x over previous
import functools

import jax
import jax.numpy as jnp
import numpy as np
from jax import lax
from jax.experimental import pallas as pl
from jax.experimental.pallas import tpu as pltpu
from jax.experimental.pallas import tpu_sc as plsc

B, N, D, V = 2, 2048, 768, 8192
MASK_ID = V
SUB, LANE = 16, 128
NO_REPLACE_PROB = np.float32(0.15)
RANDOM_TOKEN_PROB_EFF = np.float32(0.05 * (1.0 - 0.15))

TBLK = 256
VBLK = 2048
NTB, NVB = (B * N) // TBLK, V // VBLK


def _bitonic_argsort(v, ilin, lrow, si):
    idx = ilin
    for klog in range(1, 12):
        kbit = 1 << klog
        asc = (ilin & kbit) == 0
        for jlog in reversed(range(klog)):
            d = 1 << jlog
            if d < LANE:
                ax, sh = 1, d
                lower = (lrow & d) == 0
            else:
                ax, sh = 0, d // LANE
                lower = (si & sh) == 0
            pv = jnp.where(lower, jnp.roll(v, -sh, axis=ax), jnp.roll(v, sh, axis=ax))
            pidx = jnp.where(lower, jnp.roll(idx, -sh, axis=ax), jnp.roll(idx, sh, axis=ax))
            gtp = (v > pv) | ((v == pv) & (idx > pidx))
            take = gtp == (asc == lower)
            v = jnp.where(take, pv, v)
            idx = jnp.where(take, pidx, idx)
    return idx


def _halfsums(a):
    return jnp.sum(a[:, :LANE]), jnp.sum(a[:, LANE:])


def _perrow(v0, v1, rowsel0):
    return jnp.where(rowsel0, v0, v1)


def _mask_kernel(rt_ref, x_ref, perm_ref, s1_ref, s2_ref, rtok_ref,
                 mask_ref, ids_ref):
    li = lax.broadcasted_iota(jnp.int32, (SUB, 2 * LANE), 1)
    si = lax.broadcasted_iota(jnp.int32, (SUB, 2 * LANE), 0)
    lrow = li & (LANE - 1)
    rowsel0 = li < LANE
    ilin = si * LANE + lrow

    sidx = _bitonic_argsort(perm_ref[...], ilin, lrow, si)
    numtok0 = jnp.maximum((1.0 - rt_ref[0, 0, 0]) * np.float32(N), np.float32(1.0))
    numtok1 = jnp.maximum((1.0 - rt_ref[1, 0, 0]) * np.float32(N), np.float32(1.0))
    mask = sidx.astype(jnp.float32) < _perrow(numtok0, numtok1, rowsel0)

    m0, m1 = _halfsums(mask.astype(jnp.float32))
    p1 = _perrow(np.float32(N) - m0, np.float32(N) - m1, rowsel0)
    nm1 = _perrow(jnp.maximum(m0 * NO_REPLACE_PROB, np.float32(0.0)),
                  jnp.maximum(m1 * NO_REPLACE_PROB, np.float32(0.0)), rowsel0)
    a2 = jnp.where(mask, s1_ref[...], np.float32(-1.0))
    sidx2 = _bitonic_argsort(a2, ilin, lrow, si)
    norep = mask & ((sidx2.astype(jnp.float32) - p1) < nm1)
    rep = mask & (~norep)

    r0, r1 = _halfsums(rep.astype(jnp.float32))
    p2 = _perrow(np.float32(N) - r0, np.float32(N) - r1, rowsel0)
    nm2 = _perrow(jnp.maximum(r0 * RANDOM_TOKEN_PROB_EFF, np.float32(0.0)),
                  jnp.maximum(r1 * RANDOM_TOKEN_PROB_EFF, np.float32(0.0)), rowsel0)
    a3 = jnp.where(rep, s2_ref[...], np.float32(-1.0))
    sidx3 = _bitonic_argsort(a3, ilin, lrow, si)
    rndm = rep & ((sidx3.astype(jnp.float32) - p2) < nm2)
    rep_final = rep & (~rndm)

    x2 = jnp.where(rndm, rtok_ref[...], x_ref[...])
    ids_ref[...] = jnp.where(rep_final, MASK_ID, x2)
    mask_ref[...] = mask.astype(jnp.float32)


def _compute_masks(rt2, xr, pr, s1r, s2r, rtokr):
    blk = pl.BlockSpec((SUB, 2 * LANE), lambda: (0, 0))
    return pl.pallas_call(
        _mask_kernel,
        grid=(),
        in_specs=[pl.BlockSpec((B, 1, 1), lambda: (0, 0, 0), memory_space=pltpu.SMEM),
                  blk, blk, blk, blk, blk],
        out_specs=[blk, blk],
        out_shape=[
            jax.ShapeDtypeStruct((SUB, 2 * LANE), jnp.float32),
            jax.ShapeDtypeStruct((SUB, 2 * LANE), jnp.int32),
        ],
    )(rt2, xr, pr, s1r, s2r, rtokr)


_SC_CORES, _SC_SUBCORES = 2, 16
_NW = _SC_CORES * _SC_SUBCORES


def _gather_body(bpw, table_hbm, idx_hbm, out_hbm, idx_v, rows_v, sem):
    wid = lax.axis_index("s") * _SC_CORES + lax.axis_index("c")
    base = wid * bpw
    pltpu.sync_copy(idx_hbm.at[pl.ds(base, bpw)], idx_v)
    pltpu.async_copy(table_hbm.at[idx_v], rows_v, sem).wait()
    pltpu.sync_copy(rows_v, out_hbm.at[pl.ds(base, bpw)])


def _gather_rows(table, ids_chunk):
    ntok = ids_chunk.shape[0]
    bpw = ntok // _NW
    mesh = plsc.VectorSubcoreMesh(core_axis_name="c", subcore_axis_name="s")
    k = pl.kernel(
        functools.partial(_gather_body, bpw),
        out_type=jax.ShapeDtypeStruct((ntok, D), jnp.float32),
        mesh=mesh,
        scratch_types=[
            pltpu.VMEM((bpw,), jnp.int32),
            pltpu.VMEM((bpw, D), jnp.float32),
            pltpu.SemaphoreType.DMA,
        ],
    )
    return k(table, ids_chunk)


CH = 1
TPC = (B * N) // CH
NTBC = TPC // TBLK


def _loss_chunk_kernel(first, last, *refs):
    if first:
        h_ref, w_ref, lab_ref, maskw_ref, out_ref, loss_acc, cnt_acc = refs
    else:
        h_ref, w_ref, lab_ref, maskw_ref, pin_ref, out_ref, loss_acc, cnt_acc = refs
    t = pl.program_id(0)
    logits = jnp.dot(h_ref[...].astype(jnp.bfloat16), w_ref[...],
                     preferred_element_type=jnp.float32)
    m = jnp.max(logits, axis=1, keepdims=True)
    s = jnp.sum(jnp.exp(logits - m), axis=1, keepdims=True)
    hit = lax.broadcasted_iota(jnp.int32, (TBLK, V), 1) == lab_ref[...]
    lab = jnp.sum(jnp.where(hit, logits, 0.0), axis=1, keepdims=True)
    tok_ll = lab - (jnp.log(s) + m)

    @pl.when(t == 0)
    def _zero():
        if first:
            loss_acc[0, 0] = 0.0
            cnt_acc[0, 0] = 0.0
        else:
            loss_acc[0, 0] = pin_ref[0, 0]
            cnt_acc[0, 0] = pin_ref[0, 1]

    loss_acc[0, 0] += jnp.sum(maskw_ref[...] * tok_ll)
    cnt_acc[0, 0] += jnp.sum(maskw_ref[...])

    @pl.when(t == NTBC - 1)
    def _out():
        if last:
            out_ref[...] = jnp.full((1, 1), -loss_acc[0, 0] / cnt_acc[0, 0],
                                    jnp.float32)
        else:
            out_ref[0, 0] = loss_acc[0, 0]
            out_ref[0, 1] = cnt_acc[0, 0]


def _masked_loss_chunk(h, w_bf, labels, maskw, partial_in, first, last):
    in_specs = [
        pl.BlockSpec((TBLK, D), lambda t: (t, 0)),
        pl.BlockSpec((D, V), lambda t: (0, 0)),
        pl.BlockSpec((TBLK, 1), lambda t: (t, 0)),
        pl.BlockSpec((TBLK, 1), lambda t: (t, 0)),
    ]
    args = [h, w_bf, labels, maskw]
    if not first:
        in_specs.append(pl.BlockSpec((1, 2), lambda t: (0, 0),
                                     memory_space=pltpu.SMEM))
        args.append(partial_in)
    if last:
        out_spec = pl.BlockSpec((1, 1), lambda t: (0, 0))
        out_shape = jax.ShapeDtypeStruct((1, 1), jnp.float32)
    else:
        out_spec = pl.BlockSpec((1, 2), lambda t: (0, 0),
                                memory_space=pltpu.SMEM)
        out_shape = jax.ShapeDtypeStruct((1, 2), jnp.float32)
    return pl.pallas_call(
        functools.partial(_loss_chunk_kernel, first, last),
        grid=(NTBC,),
        in_specs=in_specs,
        out_specs=out_spec,
        out_shape=out_shape,
        scratch_shapes=[
            pltpu.SMEM((1, 1), jnp.float32),
            pltpu.SMEM((1, 1), jnp.float32),
        ],
    )(*args)


def kernel(x, embed, W_out, rand_times, perm_rand, subset_rand1,
           subset_rand2, random_tokens):
    def to_sxs(a):
        return a.reshape(B, SUB, LANE).transpose(1, 0, 2).reshape(SUB, B * LANE)

    def from_sxs(a):
        return a.reshape(SUB, B, LANE).transpose(1, 0, 2).reshape(B * N)

    rt2 = rand_times.reshape(B, 1, 1)
    mask_f, ids = _compute_masks(rt2, to_sxs(x), to_sxs(perm_rand),
                                 to_sxs(subset_rand1), to_sxs(subset_rand2),
                                 to_sxs(random_tokens))
    h = _gather_rows(embed, from_sxs(ids))
    w_bf = W_out.astype(jnp.bfloat16)
    out = _masked_loss_chunk(h, w_bf, x.reshape(B * N, 1),
                             from_sxs(mask_f).reshape(B * N, 1), None,
                             first=True, last=True)
    return out[0, 0]


# speedup vs baseline: 1.1549x; 1.0422x over previous
import functools

import jax
import jax.numpy as jnp
import numpy as np
from jax import lax
from jax.experimental import pallas as pl
from jax.experimental.pallas import tpu as pltpu
from jax.experimental.pallas import tpu_sc as plsc

B, N, D, V = 2, 2048, 768, 8192
MASK_ID = V
SUB, LANE = 16, 128
NO_REPLACE_PROB = np.float32(0.15)
RANDOM_TOKEN_PROB_EFF = np.float32(0.05 * (1.0 - 0.15))

TBLK = 512
VBLK = 2048
NTB, NVB = (B * N) // TBLK, V // VBLK


def _bitonic_argsort(v, ilin, lrow, si):
    idx = ilin
    for klog in range(1, 12):
        kbit = 1 << klog
        asc = (ilin & kbit) == 0
        for jlog in reversed(range(klog)):
            d = 1 << jlog
            if d < LANE:
                ax, sh = 1, d
                lower = (lrow & d) == 0
            else:
                ax, sh = 0, d // LANE
                lower = (si & sh) == 0
            pv = jnp.where(lower, jnp.roll(v, -sh, axis=ax), jnp.roll(v, sh, axis=ax))
            pidx = jnp.where(lower, jnp.roll(idx, -sh, axis=ax), jnp.roll(idx, sh, axis=ax))
            gtp = (v > pv) | ((v == pv) & (idx > pidx))
            take = gtp == (asc == lower)
            v = jnp.where(take, pv, v)
            idx = jnp.where(take, pidx, idx)
    return idx


def _halfsums(a):
    return jnp.sum(a[:, :LANE]), jnp.sum(a[:, LANE:])


def _perrow(v0, v1, rowsel0):
    return jnp.where(rowsel0, v0, v1)


def _mask_kernel(rt_ref, x_ref, perm_ref, s1_ref, s2_ref, rtok_ref,
                 mask_ref, ids_ref):
    li = lax.broadcasted_iota(jnp.int32, (SUB, 2 * LANE), 1)
    si = lax.broadcasted_iota(jnp.int32, (SUB, 2 * LANE), 0)
    lrow = li & (LANE - 1)
    rowsel0 = li < LANE
    ilin = si * LANE + lrow

    sidx = _bitonic_argsort(perm_ref[...], ilin, lrow, si)
    numtok0 = jnp.maximum((1.0 - rt_ref[0, 0, 0]) * np.float32(N), np.float32(1.0))
    numtok1 = jnp.maximum((1.0 - rt_ref[1, 0, 0]) * np.float32(N), np.float32(1.0))
    mask = sidx.astype(jnp.float32) < _perrow(numtok0, numtok1, rowsel0)

    m0, m1 = _halfsums(mask.astype(jnp.float32))
    p1 = _perrow(np.float32(N) - m0, np.float32(N) - m1, rowsel0)
    nm1 = _perrow(jnp.maximum(m0 * NO_REPLACE_PROB, np.float32(0.0)),
                  jnp.maximum(m1 * NO_REPLACE_PROB, np.float32(0.0)), rowsel0)
    a2 = jnp.where(mask, s1_ref[...], np.float32(-1.0))
    sidx2 = _bitonic_argsort(a2, ilin, lrow, si)
    norep = mask & ((sidx2.astype(jnp.float32) - p1) < nm1)
    rep = mask & (~norep)

    r0, r1 = _halfsums(rep.astype(jnp.float32))
    p2 = _perrow(np.float32(N) - r0, np.float32(N) - r1, rowsel0)
    nm2 = _perrow(jnp.maximum(r0 * RANDOM_TOKEN_PROB_EFF, np.float32(0.0)),
                  jnp.maximum(r1 * RANDOM_TOKEN_PROB_EFF, np.float32(0.0)), rowsel0)
    a3 = jnp.where(rep, s2_ref[...], np.float32(-1.0))
    sidx3 = _bitonic_argsort(a3, ilin, lrow, si)
    rndm = rep & ((sidx3.astype(jnp.float32) - p2) < nm2)
    rep_final = rep & (~rndm)

    x2 = jnp.where(rndm, rtok_ref[...], x_ref[...])
    ids_ref[...] = jnp.where(rep_final, MASK_ID, x2)
    mask_ref[...] = mask.astype(jnp.float32)


def _compute_masks(rt2, xr, pr, s1r, s2r, rtokr):
    blk = pl.BlockSpec((SUB, 2 * LANE), lambda: (0, 0))
    return pl.pallas_call(
        _mask_kernel,
        grid=(),
        in_specs=[pl.BlockSpec((B, 1, 1), lambda: (0, 0, 0), memory_space=pltpu.SMEM),
                  blk, blk, blk, blk, blk],
        out_specs=[blk, blk],
        out_shape=[
            jax.ShapeDtypeStruct((SUB, 2 * LANE), jnp.float32),
            jax.ShapeDtypeStruct((SUB, 2 * LANE), jnp.int32),
        ],
    )(rt2, xr, pr, s1r, s2r, rtokr)


_SC_CORES, _SC_SUBCORES = 2, 16
_NW = _SC_CORES * _SC_SUBCORES


def _gather_body(bpw, table_hbm, idx_hbm, out_hbm, idx_v, rows_v, sem):
    wid = lax.axis_index("s") * _SC_CORES + lax.axis_index("c")
    base = wid * bpw
    pltpu.sync_copy(idx_hbm.at[pl.ds(base, bpw)], idx_v)
    pltpu.async_copy(table_hbm.at[idx_v], rows_v, sem).wait()
    pltpu.sync_copy(rows_v, out_hbm.at[pl.ds(base, bpw)])


def _gather_rows(table, ids_chunk):
    ntok = ids_chunk.shape[0]
    bpw = ntok // _NW
    mesh = plsc.VectorSubcoreMesh(core_axis_name="c", subcore_axis_name="s")
    k = pl.kernel(
        functools.partial(_gather_body, bpw),
        out_type=jax.ShapeDtypeStruct((ntok, D), jnp.float32),
        mesh=mesh,
        scratch_types=[
            pltpu.VMEM((bpw,), jnp.int32),
            pltpu.VMEM((bpw, D), jnp.float32),
            pltpu.SemaphoreType.DMA,
        ],
    )
    return k(table, ids_chunk)


CH = 1
TPC = (B * N) // CH
NTBC = TPC // TBLK


def _loss_chunk_kernel(first, last, *refs):
    if first:
        h_ref, w_ref, lab_ref, maskw_ref, out_ref, loss_acc, cnt_acc = refs
    else:
        h_ref, w_ref, lab_ref, maskw_ref, pin_ref, out_ref, loss_acc, cnt_acc = refs
    t = pl.program_id(0)
    logits = jnp.dot(h_ref[...].astype(jnp.bfloat16), w_ref[...],
                     preferred_element_type=jnp.float32)
    m = jnp.max(logits, axis=1, keepdims=True)
    s = jnp.sum(jnp.exp(logits - m), axis=1, keepdims=True)
    hit = lax.broadcasted_iota(jnp.int32, (TBLK, V), 1) == lab_ref[...]
    lab = jnp.sum(jnp.where(hit, logits, 0.0), axis=1, keepdims=True)
    tok_ll = lab - (jnp.log(s) + m)

    @pl.when(t == 0)
    def _zero():
        if first:
            loss_acc[0, 0] = 0.0
            cnt_acc[0, 0] = 0.0
        else:
            loss_acc[0, 0] = pin_ref[0, 0]
            cnt_acc[0, 0] = pin_ref[0, 1]

    loss_acc[0, 0] += jnp.sum(maskw_ref[...] * tok_ll)
    cnt_acc[0, 0] += jnp.sum(maskw_ref[...])

    @pl.when(t == NTBC - 1)
    def _out():
        if last:
            out_ref[...] = jnp.full((1, 1), -loss_acc[0, 0] / cnt_acc[0, 0],
                                    jnp.float32)
        else:
            out_ref[0, 0] = loss_acc[0, 0]
            out_ref[0, 1] = cnt_acc[0, 0]


def _masked_loss_chunk(h, w_bf, labels, maskw, partial_in, first, last):
    in_specs = [
        pl.BlockSpec((TBLK, D), lambda t: (t, 0)),
        pl.BlockSpec((D, V), lambda t: (0, 0)),
        pl.BlockSpec((TBLK, 1), lambda t: (t, 0)),
        pl.BlockSpec((TBLK, 1), lambda t: (t, 0)),
    ]
    args = [h, w_bf, labels, maskw]
    if not first:
        in_specs.append(pl.BlockSpec((1, 2), lambda t: (0, 0),
                                     memory_space=pltpu.SMEM))
        args.append(partial_in)
    if last:
        out_spec = pl.BlockSpec((1, 1), lambda t: (0, 0))
        out_shape = jax.ShapeDtypeStruct((1, 1), jnp.float32)
    else:
        out_spec = pl.BlockSpec((1, 2), lambda t: (0, 0),
                                memory_space=pltpu.SMEM)
        out_shape = jax.ShapeDtypeStruct((1, 2), jnp.float32)
    return pl.pallas_call(
        functools.partial(_loss_chunk_kernel, first, last),
        grid=(NTBC,),
        in_specs=in_specs,
        out_specs=out_spec,
        out_shape=out_shape,
        scratch_shapes=[
            pltpu.SMEM((1, 1), jnp.float32),
            pltpu.SMEM((1, 1), jnp.float32),
        ],
    )(*args)


def kernel(x, embed, W_out, rand_times, perm_rand, subset_rand1,
           subset_rand2, random_tokens):
    def to_sxs(a):
        return a.reshape(B, SUB, LANE).transpose(1, 0, 2).reshape(SUB, B * LANE)

    def from_sxs(a):
        return a.reshape(SUB, B, LANE).transpose(1, 0, 2).reshape(B * N)

    rt2 = rand_times.reshape(B, 1, 1)
    mask_f, ids = _compute_masks(rt2, to_sxs(x), to_sxs(perm_rand),
                                 to_sxs(subset_rand1), to_sxs(subset_rand2),
                                 to_sxs(random_tokens))
    h = _gather_rows(embed, from_sxs(ids))
    w_bf = W_out.astype(jnp.bfloat16)
    out = _masked_loss_chunk(h, w_bf, x.reshape(B * N, 1),
                             from_sxs(mask_f).reshape(B * N, 1), None,
                             first=True, last=True)
    return out[0, 0]


# speedup vs baseline: 1.1646x; 1.0084x over previous
import functools

import jax
import jax.numpy as jnp
import numpy as np
from jax import lax
from jax.experimental import pallas as pl
from jax.experimental.pallas import tpu as pltpu
from jax.experimental.pallas import tpu_sc as plsc

B, N, D, V = 2, 2048, 768, 8192
MASK_ID = V
SUB, LANE = 16, 128
NO_REPLACE_PROB = np.float32(0.15)
RANDOM_TOKEN_PROB_EFF = np.float32(0.05 * (1.0 - 0.15))

TBLK = 1024
VBLK = 2048
NTB, NVB = (B * N) // TBLK, V // VBLK


def _bitonic_argsort(v, ilin, lrow, si):
    idx = ilin
    for klog in range(1, 12):
        kbit = 1 << klog
        asc = (ilin & kbit) == 0
        for jlog in reversed(range(klog)):
            d = 1 << jlog
            if d < LANE:
                ax, sh = 1, d
                lower = (lrow & d) == 0
            else:
                ax, sh = 0, d // LANE
                lower = (si & sh) == 0
            pv = jnp.where(lower, jnp.roll(v, -sh, axis=ax), jnp.roll(v, sh, axis=ax))
            pidx = jnp.where(lower, jnp.roll(idx, -sh, axis=ax), jnp.roll(idx, sh, axis=ax))
            gtp = (v > pv) | ((v == pv) & (idx > pidx))
            take = gtp == (asc == lower)
            v = jnp.where(take, pv, v)
            idx = jnp.where(take, pidx, idx)
    return idx


def _halfsums(a):
    return jnp.sum(a[:, :LANE]), jnp.sum(a[:, LANE:])


def _perrow(v0, v1, rowsel0):
    return jnp.where(rowsel0, v0, v1)


def _mask_kernel(rt_ref, x_ref, perm_ref, s1_ref, s2_ref, rtok_ref,
                 mask_ref, ids_ref):
    li = lax.broadcasted_iota(jnp.int32, (SUB, 2 * LANE), 1)
    si = lax.broadcasted_iota(jnp.int32, (SUB, 2 * LANE), 0)
    lrow = li & (LANE - 1)
    rowsel0 = li < LANE
    ilin = si * LANE + lrow

    sidx = _bitonic_argsort(perm_ref[...], ilin, lrow, si)
    numtok0 = jnp.maximum((1.0 - rt_ref[0, 0, 0]) * np.float32(N), np.float32(1.0))
    numtok1 = jnp.maximum((1.0 - rt_ref[1, 0, 0]) * np.float32(N), np.float32(1.0))
    mask = sidx.astype(jnp.float32) < _perrow(numtok0, numtok1, rowsel0)

    m0, m1 = _halfsums(mask.astype(jnp.float32))
    p1 = _perrow(np.float32(N) - m0, np.float32(N) - m1, rowsel0)
    nm1 = _perrow(jnp.maximum(m0 * NO_REPLACE_PROB, np.float32(0.0)),
                  jnp.maximum(m1 * NO_REPLACE_PROB, np.float32(0.0)), rowsel0)
    a2 = jnp.where(mask, s1_ref[...], np.float32(-1.0))
    sidx2 = _bitonic_argsort(a2, ilin, lrow, si)
    norep = mask & ((sidx2.astype(jnp.float32) - p1) < nm1)
    rep = mask & (~norep)

    r0, r1 = _halfsums(rep.astype(jnp.float32))
    p2 = _perrow(np.float32(N) - r0, np.float32(N) - r1, rowsel0)
    nm2 = _perrow(jnp.maximum(r0 * RANDOM_TOKEN_PROB_EFF, np.float32(0.0)),
                  jnp.maximum(r1 * RANDOM_TOKEN_PROB_EFF, np.float32(0.0)), rowsel0)
    a3 = jnp.where(rep, s2_ref[...], np.float32(-1.0))
    sidx3 = _bitonic_argsort(a3, ilin, lrow, si)
    rndm = rep & ((sidx3.astype(jnp.float32) - p2) < nm2)
    rep_final = rep & (~rndm)

    x2 = jnp.where(rndm, rtok_ref[...], x_ref[...])
    ids_ref[...] = jnp.where(rep_final, MASK_ID, x2)
    mask_ref[...] = mask.astype(jnp.float32)


def _compute_masks(rt2, xr, pr, s1r, s2r, rtokr):
    blk = pl.BlockSpec((SUB, 2 * LANE), lambda: (0, 0))
    return pl.pallas_call(
        _mask_kernel,
        grid=(),
        in_specs=[pl.BlockSpec((B, 1, 1), lambda: (0, 0, 0), memory_space=pltpu.SMEM),
                  blk, blk, blk, blk, blk],
        out_specs=[blk, blk],
        out_shape=[
            jax.ShapeDtypeStruct((SUB, 2 * LANE), jnp.float32),
            jax.ShapeDtypeStruct((SUB, 2 * LANE), jnp.int32),
        ],
    )(rt2, xr, pr, s1r, s2r, rtokr)


_SC_CORES, _SC_SUBCORES = 2, 16
_NW = _SC_CORES * _SC_SUBCORES


def _gather_body(bpw, table_hbm, idx_hbm, out_hbm, idx_v, rows_v, sem):
    wid = lax.axis_index("s") * _SC_CORES + lax.axis_index("c")
    base = wid * bpw
    pltpu.sync_copy(idx_hbm.at[pl.ds(base, bpw)], idx_v)
    pltpu.async_copy(table_hbm.at[idx_v], rows_v, sem).wait()
    pltpu.sync_copy(rows_v, out_hbm.at[pl.ds(base, bpw)])


def _gather_rows(table, ids_chunk):
    ntok = ids_chunk.shape[0]
    bpw = ntok // _NW
    mesh = plsc.VectorSubcoreMesh(core_axis_name="c", subcore_axis_name="s")
    k = pl.kernel(
        functools.partial(_gather_body, bpw),
        out_type=jax.ShapeDtypeStruct((ntok, D), jnp.float32),
        mesh=mesh,
        scratch_types=[
            pltpu.VMEM((bpw,), jnp.int32),
            pltpu.VMEM((bpw, D), jnp.float32),
            pltpu.SemaphoreType.DMA,
        ],
    )
    return k(table, ids_chunk)


CH = 1
TPC = (B * N) // CH
NTBC = TPC // TBLK


def _loss_chunk_kernel(first, last, *refs):
    if first:
        h_ref, w_ref, lab_ref, maskw_ref, out_ref, loss_acc, cnt_acc = refs
    else:
        h_ref, w_ref, lab_ref, maskw_ref, pin_ref, out_ref, loss_acc, cnt_acc = refs
    t = pl.program_id(0)
    logits = jnp.dot(h_ref[...].astype(jnp.bfloat16), w_ref[...],
                     preferred_element_type=jnp.float32)
    m = jnp.max(logits, axis=1, keepdims=True)
    s = jnp.sum(jnp.exp(logits - m), axis=1, keepdims=True)
    hit = lax.broadcasted_iota(jnp.int32, (TBLK, V), 1) == lab_ref[...]
    lab = jnp.sum(jnp.where(hit, logits, 0.0), axis=1, keepdims=True)
    tok_ll = lab - (jnp.log(s) + m)

    @pl.when(t == 0)
    def _zero():
        if first:
            loss_acc[0, 0] = 0.0
            cnt_acc[0, 0] = 0.0
        else:
            loss_acc[0, 0] = pin_ref[0, 0]
            cnt_acc[0, 0] = pin_ref[0, 1]

    loss_acc[0, 0] += jnp.sum(maskw_ref[...] * tok_ll)
    cnt_acc[0, 0] += jnp.sum(maskw_ref[...])

    @pl.when(t == NTBC - 1)
    def _out():
        if last:
            out_ref[...] = jnp.full((1, 1), -loss_acc[0, 0] / cnt_acc[0, 0],
                                    jnp.float32)
        else:
            out_ref[0, 0] = loss_acc[0, 0]
            out_ref[0, 1] = cnt_acc[0, 0]


def _masked_loss_chunk(h, w_bf, labels, maskw, partial_in, first, last):
    in_specs = [
        pl.BlockSpec((TBLK, D), lambda t: (t, 0)),
        pl.BlockSpec((D, V), lambda t: (0, 0)),
        pl.BlockSpec((TBLK, 1), lambda t: (t, 0)),
        pl.BlockSpec((TBLK, 1), lambda t: (t, 0)),
    ]
    args = [h, w_bf, labels, maskw]
    if not first:
        in_specs.append(pl.BlockSpec((1, 2), lambda t: (0, 0),
                                     memory_space=pltpu.SMEM))
        args.append(partial_in)
    if last:
        out_spec = pl.BlockSpec((1, 1), lambda t: (0, 0))
        out_shape = jax.ShapeDtypeStruct((1, 1), jnp.float32)
    else:
        out_spec = pl.BlockSpec((1, 2), lambda t: (0, 0),
                                memory_space=pltpu.SMEM)
        out_shape = jax.ShapeDtypeStruct((1, 2), jnp.float32)
    return pl.pallas_call(
        functools.partial(_loss_chunk_kernel, first, last),
        grid=(NTBC,),
        in_specs=in_specs,
        out_specs=out_spec,
        out_shape=out_shape,
        scratch_shapes=[
            pltpu.SMEM((1, 1), jnp.float32),
            pltpu.SMEM((1, 1), jnp.float32),
        ],
    )(*args)


def kernel(x, embed, W_out, rand_times, perm_rand, subset_rand1,
           subset_rand2, random_tokens):
    def to_sxs(a):
        return a.reshape(B, SUB, LANE).transpose(1, 0, 2).reshape(SUB, B * LANE)

    def from_sxs(a):
        return a.reshape(SUB, B, LANE).transpose(1, 0, 2).reshape(B * N)

    rt2 = rand_times.reshape(B, 1, 1)
    mask_f, ids = _compute_masks(rt2, to_sxs(x), to_sxs(perm_rand),
                                 to_sxs(subset_rand1), to_sxs(subset_rand2),
                                 to_sxs(random_tokens))
    h = _gather_rows(embed, from_sxs(ids))
    w_bf = W_out.astype(jnp.bfloat16)
    out = _masked_loss_chunk(h, w_bf, x.reshape(B * N, 1),
                             from_sxs(mask_f).reshape(B * N, 1), None,
                             first=True, last=True)
    return out[0, 0]
